# baseline (speedup 1.0000x reference)
import functools

import jax
import jax.numpy as jnp
from jax import lax
from jax.experimental import pallas as pl
from jax.experimental.pallas import tpu as pltpu
from jax.experimental.pallas import tpu_sc as plsc

B, S, D, V = 4096, 200, 64, 1000000
NC, NS = 2, 16
NW = NC * NS
BT = B // 128
FT = D // 8
LANES = 16
SC_P = 2
NCHUNK = S // SC_P

_mesh = plsc.VectorSubcoreMesh(core_axis_name="c", subcore_axis_name="s")


@functools.partial(
    pl.kernel,
    out_type=jax.ShapeDtypeStruct((S, FT, BT, 8, 128), jnp.float32),
    mesh=_mesh,
    scratch_types=[
        pltpu.VMEM((S, 128), jnp.int32),
        pltpu.VMEM((2, SC_P * 128, D), jnp.float32),
        pltpu.VMEM((2, SC_P, FT, 8, 128), jnp.float32),
        pltpu.VMEM((2, SC_P, D * LANES), jnp.float32),
        pltpu.SemaphoreType.DMA,
        pltpu.SemaphoreType.DMA,
        pltpu.SemaphoreType.DMA,
        pltpu.SemaphoreType.DMA,
    ],
    compiler_params=pltpu.CompilerParams(use_tc_tiling_on_sc=False,
                                         needs_layout_passes=False),
)
def _emb(idxt_hbm, table_hbm, posb_hbm, out_hbm,
         idx_v, rows_v, outt_v, posb_v, gsem0, gsem1, wsem0, wsem1):
    w = lax.axis_index("s") * NC + lax.axis_index("c")
    gsems = (gsem0, gsem1)
    wsems = (wsem0, wsem1)

    pltpu.sync_copy(idxt_hbm.at[:, pl.ds(pl.multiple_of(w * 128, 128), 128)], idx_v)

    def fire_gather(buf, c):
        s0 = pl.multiple_of(c * SC_P, SC_P)
        pltpu.async_copy(posb_hbm.at[pl.ds(s0, SC_P)], posb_v.at[buf], gsems[buf])
        for h in range(SC_P):
            pltpu.async_copy(table_hbm.at[idx_v.at[s0 + h]],
                             rows_v.at[buf, pl.ds(h * 128, 128)], gsems[buf])

    def wait_gather(buf, c):
        s0 = pl.multiple_of(c * SC_P, SC_P)
        pltpu.make_async_copy(posb_hbm.at[pl.ds(s0, SC_P)], posb_v.at[buf],
                              gsems[buf]).wait()
        for h in range(SC_P):
            pltpu.make_async_copy(table_hbm.at[idx_v.at[s0 + h]],
                                  rows_v.at[buf, pl.ds(h * 128, 128)],
                                  gsems[buf]).wait()

    def fire_write(buf, c):
        s0 = pl.multiple_of(c * SC_P, SC_P)
        pltpu.async_copy(outt_v.at[buf], out_hbm.at[pl.ds(s0, SC_P), :, w],
                         wsems[buf])

    def wait_write(buf, c):
        s0 = pl.multiple_of(c * SC_P, SC_P)
        pltpu.make_async_copy(outt_v.at[buf], out_hbm.at[pl.ds(s0, SC_P), :, w],
                              wsems[buf]).wait()

    lane_iota = lax.iota(jnp.int32, LANES)

    def transpose_add(buf):
        for h in range(SC_P):
            for ft in range(FT):
                pvs = [posb_v[buf, h, pl.ds((ft * 8 + fs) * LANES, LANES)]
                       for fs in range(8)]

                @plsc.parallel_loop(0, 128 // LANES, unroll=4)
                def _tl(tl, _h=h, _ft=ft, _pvs=pvs):
                    t16 = _h * 128 + tl * LANES + lane_iota
                    for fs in range(8):
                        col = jnp.full((LANES,), _ft * 8 + fs, jnp.int32)
                        v = plsc.load_gather(rows_v.at[buf], [t16, col])
                        outt_v[buf, _h, _ft, fs, pl.ds(tl * LANES, LANES)] = (
                            v + _pvs[fs])

    fire_gather(0, 0)
    fire_gather(1, 1)

    def body2(i, _):
        for buf in (0, 1):
            c = 2 * i + buf
            wait_gather(buf, c)

            @pl.when(i > 0)
            def _():
                wait_write(buf, c - 2)

            transpose_add(buf)
            fire_write(buf, c)

            @pl.when(c + 2 < NCHUNK)
            def _():
                fire_gather(buf, c + 2)

        return 0

    lax.fori_loop(0, NCHUNK // 2, body2, 0)
    wait_write(0, NCHUNK - 2)
    wait_write(1, NCHUNK - 1)


def kernel(inputs, token_table, pos_table):
    idxt = inputs.astype(jnp.int32).T
    posb = jnp.broadcast_to(
        pos_table[:, :, None], (S, D, LANES)).reshape(S, D * LANES)
    out = _emb(idxt, token_table, posb)
    return out.transpose(2, 4, 0, 1, 3).reshape(B, S, D)


# speedup vs baseline: 2.1298x; 2.1298x over previous
import functools

import jax
import jax.numpy as jnp
from jax import lax
from jax.experimental import pallas as pl
from jax.experimental.pallas import tpu as pltpu
from jax.experimental.pallas import tpu_sc as plsc

B, S, D, V = 4096, 200, 64, 1000000
NC, NS = 2, 16
NW = NC * NS
BT = B // 128
FT = D // 8
LANES = 16
SC_P = 2
NCHUNK = S // SC_P
STRIDE = 129

_mesh = plsc.VectorSubcoreMesh(core_axis_name="c", subcore_axis_name="s")


@functools.partial(
    pl.kernel,
    out_type=jax.ShapeDtypeStruct((S, FT, BT, 8, 128), jnp.float32),
    mesh=_mesh,
    scratch_types=[
        pltpu.VMEM((S, 128), jnp.int32),
        pltpu.VMEM((2, SC_P * 128, D), jnp.float32),
        pltpu.VMEM((2, SC_P, FT, 8, STRIDE), jnp.float32),
        pltpu.VMEM((2, SC_P, D), jnp.float32),
        pltpu.SemaphoreType.DMA,
        pltpu.SemaphoreType.DMA,
        pltpu.SemaphoreType.DMA,
        pltpu.SemaphoreType.DMA,
    ],
    compiler_params=pltpu.CompilerParams(use_tc_tiling_on_sc=False,
                                         needs_layout_passes=False),
)
def _emb(idxt_hbm, table_hbm, pos_hbm, out_hbm,
         idx_v, rows_v, outt_v, pos_v, gsem0, gsem1, wsem0, wsem1):
    w = lax.axis_index("s") * NC + lax.axis_index("c")
    gsems = (gsem0, gsem1)
    wsems = (wsem0, wsem1)

    pltpu.sync_copy(idxt_hbm.at[:, pl.ds(pl.multiple_of(w * 128, 128), 128)], idx_v)

    def fire_gather(buf, c):
        s0 = pl.multiple_of(c * SC_P, SC_P)
        pltpu.async_copy(pos_hbm.at[pl.ds(s0, SC_P)], pos_v.at[buf], gsems[buf])
        for h in range(SC_P):
            pltpu.async_copy(table_hbm.at[idx_v.at[s0 + h]],
                             rows_v.at[buf, pl.ds(h * 128, 128)], gsems[buf])

    def wait_gather(buf, c):
        s0 = pl.multiple_of(c * SC_P, SC_P)
        pltpu.make_async_copy(pos_hbm.at[pl.ds(s0, SC_P)], pos_v.at[buf],
                              gsems[buf]).wait()
        for h in range(SC_P):
            pltpu.make_async_copy(table_hbm.at[idx_v.at[s0 + h]],
                                  rows_v.at[buf, pl.ds(h * 128, 128)],
                                  gsems[buf]).wait()

    def fire_write(buf, c):
        s0 = pl.multiple_of(c * SC_P, SC_P)
        pltpu.async_copy(outt_v.at[buf, :, :, :, pl.ds(0, 128)],
                         out_hbm.at[pl.ds(s0, SC_P), :, w], wsems[buf])

    def wait_write(buf, c):
        s0 = pl.multiple_of(c * SC_P, SC_P)
        pltpu.make_async_copy(outt_v.at[buf, :, :, :, pl.ds(0, 128)],
                              out_hbm.at[pl.ds(s0, SC_P), :, w],
                              wsems[buf]).wait()

    lane_iota = lax.iota(jnp.int32, LANES)
    ftv = [(jnp.int32(c * 16) + lane_iota) >> 3 for c in range(D // LANES)]
    fsv = [(jnp.int32(c * 16) + lane_iota) & 7 for c in range(D // LANES)]

    def transpose_add(buf):
        for h in range(SC_P):
            pvec = [pos_v[buf, h, pl.ds(c * LANES, LANES)]
                    for c in range(D // LANES)]
            dst = outt_v.at[buf, h]

            @plsc.parallel_loop(0, 128, unroll=4)
            def _t(t, _h=h, _pvec=pvec, _dst=dst):
                tv = jnp.full((LANES,), t, jnp.int32)
                for c in range(D // LANES):
                    v = rows_v[buf, _h * 128 + t, pl.ds(c * LANES, LANES)]
                    plsc.store_scatter(_dst, [ftv[c], fsv[c], tv],
                                       v + _pvec[c])

    fire_gather(0, 0)
    fire_gather(1, 1)

    def body2(i, _):
        for buf in (0, 1):
            c = 2 * i + buf
            wait_gather(buf, c)

            @pl.when(i > 0)
            def _():
                wait_write(buf, c - 2)

            transpose_add(buf)
            fire_write(buf, c)

            @pl.when(c + 2 < NCHUNK)
            def _():
                fire_gather(buf, c + 2)

        return 0

    lax.fori_loop(0, NCHUNK // 2, body2, 0)
    wait_write(0, NCHUNK - 2)
    wait_write(1, NCHUNK - 1)


def kernel(inputs, token_table, pos_table):
    idxt = inputs.astype(jnp.int32).T
    out = _emb(idxt, token_table, pos_table)
    return out.transpose(2, 4, 0, 1, 3).reshape(B, S, D)
